# baseline (speedup 1.0000x reference)
import functools

import jax
import jax.numpy as jnp
from jax import lax
from jax.experimental import pallas as pl
from jax.experimental.pallas import tpu as pltpu
from jax.experimental.pallas import tpu_sc as plsc

N = 10000
E = 320000
IN_DIM = 128
HID = 512
OUT_DIM = 128

NC = 2
NS = 16
NW = NC * NS
CH = 128
CPW = 80
HPW = CPW // 2
EPAD = NW * CPW * CH
PAD_NODE = N
G_ROWS = N + 16
NROWS = 10112
RPT = NROWS // NS
F = 128
DEG_W = 128
NB = 2
NG = HPW // NB


@functools.cache
def _mesh():
  return plsc.VectorSubcoreMesh(
      core_axis_name="c", subcore_axis_name="s", num_cores=NC, num_subcores=NS)


CPS = 2 * CPW
X0 = 40


def _agg_body(g, srcb, dstb, zeros, pp, acc, src_v, dst_v, rows, gsems, ssems):
  c = lax.axis_index("c")
  s = lax.axis_index("s")
  pltpu.sync_copy(zeros.at[pl.ds(s * RPT, RPT)], acc.at[pl.ds(s * RPT, RPT)])
  plsc.subcore_barrier()

  nstg = jnp.where(c == 0, X0 // HPW, (CPS - X0) // HPW)
  base0 = s * CPS + jnp.where(c == 0, 0, X0)

  def stage(st, carry):
    base = base0 + st * HPW
    pltpu.sync_copy(srcb.at[pl.ds(base, HPW)], src_v)
    pltpu.sync_copy(dstb.at[pl.ds(base, HPW)], dst_v)

    def group(gi, carry2):
      gdescs = [
          pltpu.async_copy(g.at[src_v.at[gi * NB + b]], rows[b], gsems[b])
          for b in range(NB)
      ]
      sdescs = []
      for b in range(NB):
        gdescs[b].wait()
        sdescs.append(
            pltpu.async_copy(rows[b], acc.at[dst_v.at[gi * NB + b]],
                             ssems[b], add=True))
      for b in range(NB):
        sdescs[b].wait()
      return carry2

    lax.fori_loop(0, NG, group, 0)
    return carry

  lax.fori_loop(0, nstg, stage, 0)
  plsc.subcore_barrier()
  pltpu.sync_copy(acc.at[pl.ds(s * RPT, RPT)],
                  pp.at[c].at[pl.ds(s * RPT, RPT)])


@functools.cache
def _sc_agg():
  return pl.kernel(
      _agg_body,
      out_type=jax.ShapeDtypeStruct((NC, NROWS, F), jnp.float32),
      mesh=_mesh(),
      scratch_types=[
          pltpu.VMEM_SHARED((NROWS, F), jnp.float32),
          pltpu.VMEM((HPW, CH), jnp.int32),
          pltpu.VMEM((HPW, CH), jnp.int32),
          [pltpu.VMEM((CH, F), jnp.float32) for _ in range(NB)],
          [pltpu.SemaphoreType.DMA for _ in range(NB)],
          [pltpu.SemaphoreType.DMA for _ in range(NB)],
      ],
  )


_DK = 2


def _deg_body(dstb, zeros, ones, pp, acc, dst_v, ones_v, sems):
  c = lax.axis_index("c")
  s = lax.axis_index("s")
  pltpu.sync_copy(zeros.at[pl.ds(s * RPT, RPT)], acc.at[pl.ds(s * RPT, RPT)])
  ncw = jnp.where(c == 0, X0, CPS - X0)
  base = s * CPS + jnp.where(c == 0, 0, X0)
  pltpu.sync_copy(dstb.at[pl.ds(base, CPS - X0)], dst_v)
  pltpu.sync_copy(ones, ones_v)
  plsc.subcore_barrier()

  def step(j, carry):
    pltpu.sync_copy(ones_v, acc.at[dst_v.at[j]], add=True)
    return carry

  lax.fori_loop(0, ncw, step, 0)
  plsc.subcore_barrier()
  pltpu.sync_copy(acc.at[pl.ds(s * RPT, RPT)],
                  pp.at[c].at[pl.ds(s * RPT, RPT)])


@functools.cache
def _sc_deg():
  return pl.kernel(
      _deg_body,
      out_type=jax.ShapeDtypeStruct((NC, NROWS, DEG_W), jnp.float32),
      mesh=_mesh(),
      scratch_types=[
          pltpu.VMEM_SHARED((NROWS, DEG_W), jnp.float32),
          pltpu.VMEM((CPS - X0, CH), jnp.int32),
          pltpu.VMEM((CH, DEG_W), jnp.float32),
          [pltpu.SemaphoreType.DMA for _ in range(_DK)],
      ],
  )


_RB = 1000
_GRID = N // _RB


def _dinv_blk(pdeg_ref):
  deg = pdeg_ref[0, :, 0:1] + pdeg_ref[1, :, 0:1] + 1.0
  return lax.rsqrt(deg)


def _psum(p):
  return p[0] + p[1]


def _g0_body(pdeg, x, g0):
  g0[...] = x[...] * _dinv_blk(pdeg)


def _s2_body(pdeg, p, g0, w1, b1, g1):
  dinv = _dinv_blk(pdeg)
  t = dinv * (_psum(p) + g0[...])
  h = jnp.dot(t, w1[...], preferred_element_type=jnp.float32) + b1[...]
  g1[...] = dinv * jnp.maximum(h, 0.0)


def _s3_body(pdeg, p0, p1, p2, p3, g1, w2, b2, w3, gu):
  dinv = _dinv_blk(pdeg)
  agg = jnp.concatenate(
      [_psum(p0), _psum(p1), _psum(p2), _psum(p3)], axis=1) + g1[...]
  t = dinv * agg
  h = jnp.maximum(
      jnp.dot(t, w2[...], preferred_element_type=jnp.float32) + b2[...], 0.0)
  u = jnp.dot(h, w3[...], preferred_element_type=jnp.float32)
  gu[...] = dinv * u


def _s4_body(pdeg, p, gu, b3, out):
  dinv = _dinv_blk(pdeg)
  out[...] = dinv * (_psum(p) + gu[...]) + b3[...]


def _pblock(width):
  return pl.BlockSpec((NC, _RB, width), lambda i: (0, i, 0))


def _rblock(width):
  return pl.BlockSpec((_RB, width), lambda i: (i, 0))


def _wblock(r, c):
  return pl.BlockSpec((r, c), lambda i: (0, 0))


_tc_g0 = pl.pallas_call(
    _g0_body,
    grid=(_GRID,),
    in_specs=[_pblock(DEG_W), _rblock(IN_DIM)],
    out_specs=_rblock(IN_DIM),
    out_shape=jax.ShapeDtypeStruct((N, IN_DIM), jnp.float32),
)

_tc_s2 = pl.pallas_call(
    _s2_body,
    grid=(_GRID,),
    in_specs=[_pblock(DEG_W), _pblock(F), _rblock(IN_DIM),
              _wblock(IN_DIM, HID), _wblock(1, HID)],
    out_specs=_rblock(HID),
    out_shape=jax.ShapeDtypeStruct((N, HID), jnp.float32),
)

_tc_s3 = pl.pallas_call(
    _s3_body,
    grid=(_GRID,),
    in_specs=[_pblock(DEG_W), _pblock(F), _pblock(F), _pblock(F),
              _pblock(F), _rblock(HID), _wblock(HID, HID), _wblock(1, HID),
              _wblock(HID, OUT_DIM)],
    out_specs=_rblock(OUT_DIM),
    out_shape=jax.ShapeDtypeStruct((N, OUT_DIM), jnp.float32),
)

_tc_s4 = pl.pallas_call(
    _s4_body,
    grid=(_GRID,),
    in_specs=[_pblock(DEG_W), _pblock(F), _rblock(OUT_DIM),
              _wblock(1, OUT_DIM)],
    out_specs=_rblock(OUT_DIM),
    out_shape=jax.ShapeDtypeStruct((N, OUT_DIM), jnp.float32),
)


def _padg(g):
  return jnp.concatenate(
      [g, jnp.zeros((G_ROWS - N, g.shape[1]), jnp.float32)], axis=0)


@jax.jit
def kernel(x, edge_index, W1, b1, W2, b2, W3, b3):
  src = edge_index[0].astype(jnp.int32)
  dst = edge_index[1].astype(jnp.int32)
  pad = jnp.full((EPAD - E,), PAD_NODE, jnp.int32)
  srcb = jnp.concatenate([src, pad]).reshape(NW * CPW, CH)
  dstb = jnp.concatenate([dst, pad]).reshape(NW * CPW, CH)
  zeros = jnp.zeros((NROWS, F), jnp.float32)
  ones_d = jnp.ones((CH, DEG_W), jnp.float32)
  b1r = b1.reshape(1, HID)
  b2r = b2.reshape(1, HID)
  b3r = b3.reshape(1, OUT_DIM)

  agg = _sc_agg()
  pdeg = _sc_deg()(dstb, zeros, ones_d)
  g0 = _tc_g0(pdeg, x)
  p1 = agg(_padg(g0), srcb, dstb, zeros)
  g1 = _tc_s2(pdeg, p1, g0, W1, b1r)
  g1c = _padg(g1).reshape(G_ROWS, 4, F).transpose(1, 0, 2)
  p2 = [agg(g1c[cc], srcb, dstb, zeros) for cc in range(4)]
  gu = _tc_s3(pdeg, p2[0], p2[1], p2[2], p2[3], g1, W2, b2r, W3)
  p3 = agg(_padg(gu), srcb, dstb, zeros)
  return _tc_s4(pdeg, p3, gu, b3r)


# speedup vs baseline: 1.4168x; 1.4168x over previous
import functools

import jax
import jax.numpy as jnp
from jax import lax
from jax.experimental import pallas as pl
from jax.experimental.pallas import tpu as pltpu
from jax.experimental.pallas import tpu_sc as plsc

N = 10000
E = 320000
IN_DIM = 128
HID = 512
OUT_DIM = 128

NC = 2
NS = 16
NW = NC * NS
CH = 128
CPW = 80
HPW = CPW // 2
EPAD = NW * CPW * CH
PAD_NODE = N
G_ROWS = N + 16
NROWS = 10112
RPT = NROWS // NS
F = 128
DEG_W = 128
NB = 2
NG = HPW // NB


@functools.cache
def _mesh():
  return plsc.VectorSubcoreMesh(
      core_axis_name="c", subcore_axis_name="s", num_cores=NC, num_subcores=NS)


CPS = 2 * CPW
X0 = 120


def _agg_body(g, srcb, dstb, zeros, pp, acc, src_v, dst_v, rows, gsems, ssems):
  c = lax.axis_index("c")
  s = lax.axis_index("s")
  pltpu.sync_copy(zeros.at[pl.ds(s * RPT, RPT)], acc.at[pl.ds(s * RPT, RPT)])
  plsc.subcore_barrier()

  nstg = jnp.where(c == 0, X0 // HPW, (CPS - X0) // HPW)
  base0 = s * CPS + jnp.where(c == 0, 0, X0)

  def stage(st, carry):
    base = base0 + st * HPW
    pltpu.sync_copy(srcb.at[pl.ds(base, HPW)], src_v)
    pltpu.sync_copy(dstb.at[pl.ds(base, HPW)], dst_v)

    def group(gi, carry2):
      gdescs = [
          pltpu.async_copy(g.at[src_v.at[gi * NB + b]], rows[b], gsems[b])
          for b in range(NB)
      ]
      sdescs = []
      for b in range(NB):
        gdescs[b].wait()
        sdescs.append(
            pltpu.async_copy(rows[b], acc.at[dst_v.at[gi * NB + b]],
                             ssems[b], add=True))
      for b in range(NB):
        sdescs[b].wait()
      return carry2

    lax.fori_loop(0, NG, group, 0)
    return carry

  lax.fori_loop(0, nstg, stage, 0)
  plsc.subcore_barrier()
  pltpu.sync_copy(acc.at[pl.ds(s * RPT, RPT)],
                  pp.at[c].at[pl.ds(s * RPT, RPT)])


@functools.cache
def _sc_agg():
  return pl.kernel(
      _agg_body,
      out_type=jax.ShapeDtypeStruct((NC, NROWS, F), jnp.float32),
      mesh=_mesh(),
      scratch_types=[
          pltpu.VMEM_SHARED((NROWS, F), jnp.float32),
          pltpu.VMEM((HPW, CH), jnp.int32),
          pltpu.VMEM((HPW, CH), jnp.int32),
          [pltpu.VMEM((CH, F), jnp.float32) for _ in range(NB)],
          [pltpu.SemaphoreType.DMA for _ in range(NB)],
          [pltpu.SemaphoreType.DMA for _ in range(NB)],
      ],
  )


_DK = 2


def _deg_body(dstb, zeros, ones, pp, acc, dst_v, ones_v, sems):
  c = lax.axis_index("c")
  s = lax.axis_index("s")
  pltpu.sync_copy(zeros.at[pl.ds(s * RPT, RPT)], acc.at[pl.ds(s * RPT, RPT)])
  pltpu.sync_copy(ones, ones_v)
  plsc.subcore_barrier()
  nstg = jnp.where(c == 0, X0 // HPW, (CPS - X0) // HPW)
  base0 = s * CPS + jnp.where(c == 0, 0, X0)

  def stage(st, carry):
    pltpu.sync_copy(dstb.at[pl.ds(base0 + st * HPW, HPW)], dst_v)

    def step(j, carry2):
      pltpu.sync_copy(ones_v, acc.at[dst_v.at[j]], add=True)
      return carry2

    lax.fori_loop(0, HPW, step, 0)
    return carry

  lax.fori_loop(0, nstg, stage, 0)
  plsc.subcore_barrier()
  pltpu.sync_copy(acc.at[pl.ds(s * RPT, RPT)],
                  pp.at[c].at[pl.ds(s * RPT, RPT)])


@functools.cache
def _sc_deg():
  return pl.kernel(
      _deg_body,
      out_type=jax.ShapeDtypeStruct((NC, NROWS, DEG_W), jnp.float32),
      mesh=_mesh(),
      scratch_types=[
          pltpu.VMEM_SHARED((NROWS, DEG_W), jnp.float32),
          pltpu.VMEM((HPW, CH), jnp.int32),
          pltpu.VMEM((CH, DEG_W), jnp.float32),
          [pltpu.SemaphoreType.DMA for _ in range(_DK)],
      ],
  )


_RB = 1000
_GRID = N // _RB


def _dinv_blk(pdeg_ref):
  deg = pdeg_ref[0, :, 0:1] + pdeg_ref[1, :, 0:1] + 1.0
  return lax.rsqrt(deg)


def _psum(p):
  return p[0] + p[1]


def _g0_body(pdeg, x, g0):
  g0[...] = x[...] * _dinv_blk(pdeg)


def _s2_body(pdeg, p, g0, w1, b1, g1):
  dinv = _dinv_blk(pdeg)
  t = dinv * (_psum(p) + g0[...])
  h = jnp.dot(t, w1[...], preferred_element_type=jnp.float32) + b1[...]
  g1[...] = dinv * jnp.maximum(h, 0.0)


def _s3_body(pdeg, p0, p1, p2, p3, g1, w2, b2, w3, gu):
  dinv = _dinv_blk(pdeg)
  agg = jnp.concatenate(
      [_psum(p0), _psum(p1), _psum(p2), _psum(p3)], axis=1) + g1[...]
  t = dinv * agg
  h = jnp.maximum(
      jnp.dot(t, w2[...], preferred_element_type=jnp.float32) + b2[...], 0.0)
  u = jnp.dot(h, w3[...], preferred_element_type=jnp.float32)
  gu[...] = dinv * u


def _s4_body(pdeg, p, gu, b3, out):
  dinv = _dinv_blk(pdeg)
  out[...] = dinv * (_psum(p) + gu[...]) + b3[...]


def _pblock(width):
  return pl.BlockSpec((NC, _RB, width), lambda i: (0, i, 0))


def _rblock(width):
  return pl.BlockSpec((_RB, width), lambda i: (i, 0))


def _wblock(r, c):
  return pl.BlockSpec((r, c), lambda i: (0, 0))


_tc_g0 = pl.pallas_call(
    _g0_body,
    grid=(_GRID,),
    in_specs=[_pblock(DEG_W), _rblock(IN_DIM)],
    out_specs=_rblock(IN_DIM),
    out_shape=jax.ShapeDtypeStruct((N, IN_DIM), jnp.float32),
)

_tc_s2 = pl.pallas_call(
    _s2_body,
    grid=(_GRID,),
    in_specs=[_pblock(DEG_W), _pblock(F), _rblock(IN_DIM),
              _wblock(IN_DIM, HID), _wblock(1, HID)],
    out_specs=_rblock(HID),
    out_shape=jax.ShapeDtypeStruct((N, HID), jnp.float32),
)

_tc_s3 = pl.pallas_call(
    _s3_body,
    grid=(_GRID,),
    in_specs=[_pblock(DEG_W), _pblock(F), _pblock(F), _pblock(F),
              _pblock(F), _rblock(HID), _wblock(HID, HID), _wblock(1, HID),
              _wblock(HID, OUT_DIM)],
    out_specs=_rblock(OUT_DIM),
    out_shape=jax.ShapeDtypeStruct((N, OUT_DIM), jnp.float32),
)

_tc_s4 = pl.pallas_call(
    _s4_body,
    grid=(_GRID,),
    in_specs=[_pblock(DEG_W), _pblock(F), _rblock(OUT_DIM),
              _wblock(1, OUT_DIM)],
    out_specs=_rblock(OUT_DIM),
    out_shape=jax.ShapeDtypeStruct((N, OUT_DIM), jnp.float32),
)


def _padg(g):
  return jnp.concatenate(
      [g, jnp.zeros((G_ROWS - N, g.shape[1]), jnp.float32)], axis=0)


@jax.jit
def kernel(x, edge_index, W1, b1, W2, b2, W3, b3):
  src = edge_index[0].astype(jnp.int32)
  dst = edge_index[1].astype(jnp.int32)
  pad = jnp.full((EPAD - E,), PAD_NODE, jnp.int32)
  srcb = jnp.concatenate([src, pad]).reshape(NW * CPW, CH)
  dstb = jnp.concatenate([dst, pad]).reshape(NW * CPW, CH)
  zeros = jnp.zeros((NROWS, F), jnp.float32)
  ones_d = jnp.ones((CH, DEG_W), jnp.float32)
  b1r = b1.reshape(1, HID)
  b2r = b2.reshape(1, HID)
  b3r = b3.reshape(1, OUT_DIM)

  agg = _sc_agg()
  pdeg = _sc_deg()(dstb, zeros, ones_d)
  g0 = _tc_g0(pdeg, x)
  p1 = agg(_padg(g0), srcb, dstb, zeros)
  g1 = _tc_s2(pdeg, p1, g0, W1, b1r)
  g1c = _padg(g1).reshape(G_ROWS, 4, F).transpose(1, 0, 2)
  p2 = [agg(g1c[cc], srcb, dstb, zeros) for cc in range(4)]
  gu = _tc_s3(pdeg, p2[0], p2[1], p2[2], p2[3], g1, W2, b2r, W3)
  p3 = agg(_padg(gu), srcb, dstb, zeros)
  return _tc_s4(pdeg, p3, gu, b3r)


# speedup vs baseline: 1.4170x; 1.0002x over previous
import functools

import jax
import jax.numpy as jnp
from jax import lax
from jax.experimental import pallas as pl
from jax.experimental.pallas import tpu as pltpu
from jax.experimental.pallas import tpu_sc as plsc

N = 10000
E = 320000
IN_DIM = 128
HID = 512
OUT_DIM = 128

NC = 2
NS = 16
NW = NC * NS
CH = 128
CPW = 80
HPW = CPW // 2
EPAD = NW * CPW * CH
PAD_NODE = N
G_ROWS = N + 16
NROWS = 10112
RPT = NROWS // NS
F = 128
DEG_W = 128
NB = 2
NG = HPW // NB


@functools.cache
def _mesh():
  return plsc.VectorSubcoreMesh(
      core_axis_name="c", subcore_axis_name="s", num_cores=NC, num_subcores=NS)


CPS = 2 * CPW
X0 = 120


def _agg_body(g, srcb, dstb, zeros, pp, acc, src_v, dst_v, rows, gsems, ssems):
  c = lax.axis_index("c")
  s = lax.axis_index("s")
  pltpu.sync_copy(zeros.at[pl.ds(s * RPT, RPT)], acc.at[pl.ds(s * RPT, RPT)])
  plsc.subcore_barrier()

  nstg = jnp.where(c == 0, X0 // HPW, (CPS - X0) // HPW)
  base0 = s * CPS + jnp.where(c == 0, 0, X0)

  def stage(st, carry):
    base = base0 + st * HPW
    pltpu.sync_copy(srcb.at[pl.ds(base, HPW)], src_v)
    pltpu.sync_copy(dstb.at[pl.ds(base, HPW)], dst_v)

    def group(gi, carry2):
      gdescs = [
          pltpu.async_copy(g.at[src_v.at[gi * NB + b]], rows[b], gsems[b])
          for b in range(NB)
      ]
      sdescs = []
      for b in range(NB):
        gdescs[b].wait()
        sdescs.append(
            pltpu.async_copy(rows[b], acc.at[dst_v.at[gi * NB + b]],
                             ssems[b], add=True))
      for b in range(NB):
        sdescs[b].wait()
      return carry2

    lax.fori_loop(0, NG, group, 0)
    return carry

  lax.fori_loop(0, nstg, stage, 0)
  plsc.subcore_barrier()
  pltpu.sync_copy(acc.at[pl.ds(s * RPT, RPT)],
                  pp.at[c].at[pl.ds(s * RPT, RPT)])


@functools.cache
def _sc_agg():
  return pl.kernel(
      _agg_body,
      out_type=jax.ShapeDtypeStruct((NC, NROWS, F), jnp.float32),
      mesh=_mesh(),
      scratch_types=[
          pltpu.VMEM_SHARED((NROWS, F), jnp.float32),
          pltpu.VMEM((HPW, CH), jnp.int32),
          pltpu.VMEM((HPW, CH), jnp.int32),
          [pltpu.VMEM((CH, F), jnp.float32) for _ in range(NB)],
          [pltpu.SemaphoreType.DMA for _ in range(NB)],
          [pltpu.SemaphoreType.DMA for _ in range(NB)],
      ],
  )


_DK = 2


def _deg_body(dstb, zeros, ones, pp, acc, dst_v, ones_v, sems):
  c = lax.axis_index("c")
  s = lax.axis_index("s")
  pltpu.sync_copy(zeros.at[pl.ds(s * RPT, RPT)], acc.at[pl.ds(s * RPT, RPT)])
  pltpu.sync_copy(ones, ones_v)
  plsc.subcore_barrier()
  nstg = jnp.where(c == 0, X0 // HPW, (CPS - X0) // HPW)
  base0 = s * CPS + jnp.where(c == 0, 0, X0)

  def stage(st, carry):
    pltpu.sync_copy(dstb.at[pl.ds(base0 + st * HPW, HPW)], dst_v)

    def group(gi, carry2):
      descs = [
          pltpu.async_copy(ones_v, acc.at[dst_v.at[gi * _DK + b]], sems[b],
                           add=True)
          for b in range(_DK)
      ]
      for d in descs:
        d.wait()
      return carry2

    lax.fori_loop(0, HPW // _DK, group, 0)
    return carry

  lax.fori_loop(0, nstg, stage, 0)
  plsc.subcore_barrier()
  pltpu.sync_copy(acc.at[pl.ds(s * RPT, RPT)],
                  pp.at[c].at[pl.ds(s * RPT, RPT)])


@functools.cache
def _sc_deg():
  return pl.kernel(
      _deg_body,
      out_type=jax.ShapeDtypeStruct((NC, NROWS, DEG_W), jnp.float32),
      mesh=_mesh(),
      scratch_types=[
          pltpu.VMEM_SHARED((NROWS, DEG_W), jnp.float32),
          pltpu.VMEM((HPW, CH), jnp.int32),
          pltpu.VMEM((CH, DEG_W), jnp.float32),
          [pltpu.SemaphoreType.DMA for _ in range(_DK)],
      ],
  )


_RB = 1000
_GRID = N // _RB


def _dinv_blk(pdeg_ref):
  deg = pdeg_ref[0, :, 0:1] + pdeg_ref[1, :, 0:1] + 1.0
  return lax.rsqrt(deg)


def _psum(p):
  return p[0] + p[1]


def _g0_body(pdeg, x, g0):
  g0[...] = x[...] * _dinv_blk(pdeg)


def _s2_body(pdeg, p, g0, w1, b1, g1):
  dinv = _dinv_blk(pdeg)
  t = dinv * (_psum(p) + g0[...])
  h = jnp.dot(t, w1[...], preferred_element_type=jnp.float32) + b1[...]
  g1[...] = dinv * jnp.maximum(h, 0.0)


def _s3_body(pdeg, p0, p1, p2, p3, g1, w2, b2, w3, gu):
  dinv = _dinv_blk(pdeg)
  agg = jnp.concatenate(
      [_psum(p0), _psum(p1), _psum(p2), _psum(p3)], axis=1) + g1[...]
  t = dinv * agg
  h = jnp.maximum(
      jnp.dot(t, w2[...], preferred_element_type=jnp.float32) + b2[...], 0.0)
  u = jnp.dot(h, w3[...], preferred_element_type=jnp.float32)
  gu[...] = dinv * u


def _s4_body(pdeg, p, gu, b3, out):
  dinv = _dinv_blk(pdeg)
  out[...] = dinv * (_psum(p) + gu[...]) + b3[...]


def _pblock(width):
  return pl.BlockSpec((NC, _RB, width), lambda i: (0, i, 0))


def _rblock(width):
  return pl.BlockSpec((_RB, width), lambda i: (i, 0))


def _wblock(r, c):
  return pl.BlockSpec((r, c), lambda i: (0, 0))


_tc_g0 = pl.pallas_call(
    _g0_body,
    grid=(_GRID,),
    in_specs=[_pblock(DEG_W), _rblock(IN_DIM)],
    out_specs=_rblock(IN_DIM),
    out_shape=jax.ShapeDtypeStruct((N, IN_DIM), jnp.float32),
)

_tc_s2 = pl.pallas_call(
    _s2_body,
    grid=(_GRID,),
    in_specs=[_pblock(DEG_W), _pblock(F), _rblock(IN_DIM),
              _wblock(IN_DIM, HID), _wblock(1, HID)],
    out_specs=_rblock(HID),
    out_shape=jax.ShapeDtypeStruct((N, HID), jnp.float32),
)

_tc_s3 = pl.pallas_call(
    _s3_body,
    grid=(_GRID,),
    in_specs=[_pblock(DEG_W), _pblock(F), _pblock(F), _pblock(F),
              _pblock(F), _rblock(HID), _wblock(HID, HID), _wblock(1, HID),
              _wblock(HID, OUT_DIM)],
    out_specs=_rblock(OUT_DIM),
    out_shape=jax.ShapeDtypeStruct((N, OUT_DIM), jnp.float32),
)

_tc_s4 = pl.pallas_call(
    _s4_body,
    grid=(_GRID,),
    in_specs=[_pblock(DEG_W), _pblock(F), _rblock(OUT_DIM),
              _wblock(1, OUT_DIM)],
    out_specs=_rblock(OUT_DIM),
    out_shape=jax.ShapeDtypeStruct((N, OUT_DIM), jnp.float32),
)


def _padg(g):
  return jnp.concatenate(
      [g, jnp.zeros((G_ROWS - N, g.shape[1]), jnp.float32)], axis=0)


@jax.jit
def kernel(x, edge_index, W1, b1, W2, b2, W3, b3):
  src = edge_index[0].astype(jnp.int32)
  dst = edge_index[1].astype(jnp.int32)
  pad = jnp.full((EPAD - E,), PAD_NODE, jnp.int32)
  srcb = jnp.concatenate([src, pad]).reshape(NW * CPW, CH)
  dstb = jnp.concatenate([dst, pad]).reshape(NW * CPW, CH)
  zeros = jnp.zeros((NROWS, F), jnp.float32)
  ones_d = jnp.ones((CH, DEG_W), jnp.float32)
  b1r = b1.reshape(1, HID)
  b2r = b2.reshape(1, HID)
  b3r = b3.reshape(1, OUT_DIM)

  agg = _sc_agg()
  pdeg = _sc_deg()(dstb, zeros, ones_d)
  g0 = _tc_g0(pdeg, x)
  p1 = agg(_padg(g0), srcb, dstb, zeros)
  g1 = _tc_s2(pdeg, p1, g0, W1, b1r)
  g1c = _padg(g1).reshape(G_ROWS, 4, F).transpose(1, 0, 2)
  p2 = [agg(g1c[cc], srcb, dstb, zeros) for cc in range(4)]
  gu = _tc_s3(pdeg, p2[0], p2[1], p2[2], p2[3], g1, W2, b2r, W3)
  p3 = agg(_padg(gu), srcb, dstb, zeros)
  return _tc_s4(pdeg, p3, gu, b3r)
